# baseline (speedup 1.0000x reference)
import functools

import jax
import jax.numpy as jnp
from jax import lax
from jax.experimental import pallas as pl
from jax.experimental.pallas import tpu as pltpu
from jax.experimental.pallas import tpu_sc as plsc

_NC = 2
_NS = 16
_NW = _NC * _NS


def _sc_add_tail(inp, pos, s_base, s_len):
    S, B, E = inp.shape
    POS_W = s_len // _NW
    CH = 8 if POS_W >= 16 else POS_W // 2
    NCHUNK = POS_W // CH
    NJ = NCHUNK // 2
    NV = E // 16

    mesh = plsc.VectorSubcoreMesh(core_axis_name="c", subcore_axis_name="s")

    @functools.partial(
        pl.kernel,
        mesh=mesh,
        out_type=jax.ShapeDtypeStruct((s_len, B, E), jnp.float32),
        scratch_types=[
            pltpu.VMEM((CH, E), jnp.float32),
            pltpu.VMEM((CH, B, E), jnp.float32),
            pltpu.VMEM((CH, E), jnp.float32),
            pltpu.VMEM((CH, B, E), jnp.float32),
            pltpu.SemaphoreType.DMA,
            pltpu.SemaphoreType.DMA,
            pltpu.SemaphoreType.DMA,
            pltpu.SemaphoreType.DMA,
            pltpu.SemaphoreType.DMA,
            pltpu.SemaphoreType.DMA,
        ],
    )
    def body(in_hbm, pos_hbm, out_hbm, pos_v0, io_v0, pos_v1, io_v1,
             psem0, isem0, osem0, psem1, isem1, osem1):
        wid = lax.axis_index("s") * _NC + lax.axis_index("c")
        base_c = wid * NCHUNK

        def in_copies(c, pos_b, io_b, psem, isem):
            pbase = s_base + c * CH
            return (
                pltpu.make_async_copy(pos_hbm.at[pl.ds(pbase, CH)], pos_b, psem),
                pltpu.make_async_copy(in_hbm.at[pl.ds(pbase, CH)], io_b, isem),
            )

        def start_in(c, pos_b, io_b, psem, isem):
            ca, cb = in_copies(c, pos_b, io_b, psem, isem)
            ca.start()
            cb.start()

        def wait_in(pos_b, io_b, psem, isem):
            ca, cb = in_copies(base_c, pos_b, io_b, psem, isem)
            ca.wait()
            cb.wait()

        def out_copy(c, io_b, osem):
            return pltpu.make_async_copy(
                io_b, out_hbm.at[pl.ds(c * CH, CH)], osem)

        def compute(pos_b, io_b):
            @plsc.parallel_loop(0, CH, unroll=2)
            def _(p):
                for v in range(NV):
                    off = v * 16
                    pv = pos_b[p, pl.ds(off, 16)]
                    for b in range(B):
                        plsc.addupdate(io_b.at[p, b, pl.ds(off, 16)], pv)

        start_in(base_c, pos_v0, io_v0, psem0, isem0)

        def pair(j, carry):
            c0 = base_c + 2 * j
            c1 = c0 + 1
            c2 = c0 + 2

            @pl.when(j > 0)
            def _():
                out_copy(c1, io_v1, osem1).wait()

            start_in(c1, pos_v1, io_v1, psem1, isem1)

            wait_in(pos_v0, io_v0, psem0, isem0)
            compute(pos_v0, io_v0)
            out_copy(c0, io_v0, osem0).start()

            wait_in(pos_v1, io_v1, psem1, isem1)
            compute(pos_v1, io_v1)
            out_copy(c1, io_v1, osem1).start()

            @pl.when(j + 1 < NJ)
            def _():
                out_copy(c0, io_v0, osem0).wait()
                start_in(c2, pos_v0, io_v0, psem0, isem0)

            return carry

        lax.fori_loop(0, NJ, pair, 0)
        out_copy(base_c, io_v0, osem0).wait()
        out_copy(base_c, io_v1, osem1).wait()

    return body(inp, pos)


def _tc_body(in_ref, pos_ref, out_ref):
    out_ref[...] = in_ref[...] + pos_ref[...][:, None, :]


def _tc_add_head(inp, pos, s_tc, bs):
    S, B, E = inp.shape
    grid = (s_tc // bs,)
    return pl.pallas_call(
        _tc_body,
        grid=grid,
        in_specs=[
            pl.BlockSpec((bs, B, E), lambda i: (i, 0, 0)),
            pl.BlockSpec((bs, E), lambda i: (i, 0)),
        ],
        out_specs=pl.BlockSpec((bs, B, E), lambda i: (i, 0, 0)),
        out_shape=jax.ShapeDtypeStruct((S, B, E), inp.dtype),
    )(inp, pos)


def kernel(input, pos_table):
    S, B, E = input.shape
    pos = pos_table[:S]
    S_SC = 512
    S_TC = S - S_SC
    sc_out = _sc_add_tail(input, pos, S_TC, S_SC)
    tc_out = _tc_add_head(input, pos, S_TC, 512)
    return lax.dynamic_update_slice(tc_out, sc_out, (S_TC, 0, 0))


# speedup vs baseline: 1.0313x; 1.0313x over previous
import functools

import jax
import jax.numpy as jnp
from jax import lax
from jax.experimental import pallas as pl
from jax.experimental.pallas import tpu as pltpu
from jax.experimental.pallas import tpu_sc as plsc

_NC = 2
_NS = 16
_NW = _NC * _NS


def _sc_add_tail(inp, pos, s_base, s_len):
    S, B, E = inp.shape
    POS_W = s_len // _NW
    CH = 8 if POS_W >= 16 else POS_W // 2
    NCHUNK = POS_W // CH
    NJ = NCHUNK // 2
    NV = E // 16

    mesh = plsc.VectorSubcoreMesh(core_axis_name="c", subcore_axis_name="s")

    @functools.partial(
        pl.kernel,
        mesh=mesh,
        out_type=jax.ShapeDtypeStruct((s_len, B, E), jnp.float32),
        scratch_types=[
            pltpu.VMEM((CH, E), jnp.float32),
            pltpu.VMEM((CH, B, E), jnp.float32),
            pltpu.VMEM((CH, E), jnp.float32),
            pltpu.VMEM((CH, B, E), jnp.float32),
            pltpu.SemaphoreType.DMA,
            pltpu.SemaphoreType.DMA,
            pltpu.SemaphoreType.DMA,
            pltpu.SemaphoreType.DMA,
            pltpu.SemaphoreType.DMA,
            pltpu.SemaphoreType.DMA,
        ],
    )
    def body(in_hbm, pos_hbm, out_hbm, pos_v0, io_v0, pos_v1, io_v1,
             psem0, isem0, osem0, psem1, isem1, osem1):
        wid = lax.axis_index("s") * _NC + lax.axis_index("c")
        base_c = wid * NCHUNK

        def in_copies(c, pos_b, io_b, psem, isem):
            pbase = s_base + c * CH
            return (
                pltpu.make_async_copy(pos_hbm.at[pl.ds(pbase, CH)], pos_b, psem),
                pltpu.make_async_copy(in_hbm.at[pl.ds(pbase, CH)], io_b, isem),
            )

        def start_in(c, pos_b, io_b, psem, isem):
            ca, cb = in_copies(c, pos_b, io_b, psem, isem)
            ca.start()
            cb.start()

        def wait_in(pos_b, io_b, psem, isem):
            ca, cb = in_copies(base_c, pos_b, io_b, psem, isem)
            ca.wait()
            cb.wait()

        def out_copy(c, io_b, osem):
            return pltpu.make_async_copy(
                io_b, out_hbm.at[pl.ds(c * CH, CH)], osem)

        def compute(pos_b, io_b):
            @plsc.parallel_loop(0, CH, unroll=2)
            def _(p):
                for v in range(NV):
                    off = v * 16
                    pv = pos_b[p, pl.ds(off, 16)]
                    for b in range(B):
                        plsc.addupdate(io_b.at[p, b, pl.ds(off, 16)], pv)

        start_in(base_c, pos_v0, io_v0, psem0, isem0)

        def pair(j, carry):
            c0 = base_c + 2 * j
            c1 = c0 + 1
            c2 = c0 + 2

            @pl.when(j > 0)
            def _():
                out_copy(c1, io_v1, osem1).wait()

            start_in(c1, pos_v1, io_v1, psem1, isem1)

            wait_in(pos_v0, io_v0, psem0, isem0)
            compute(pos_v0, io_v0)
            out_copy(c0, io_v0, osem0).start()

            wait_in(pos_v1, io_v1, psem1, isem1)
            compute(pos_v1, io_v1)
            out_copy(c1, io_v1, osem1).start()

            @pl.when(j + 1 < NJ)
            def _():
                out_copy(c0, io_v0, osem0).wait()
                start_in(c2, pos_v0, io_v0, psem0, isem0)

            return carry

        lax.fori_loop(0, NJ, pair, 0)
        out_copy(base_c, io_v0, osem0).wait()
        out_copy(base_c, io_v1, osem1).wait()

    return body(inp, pos)


def _tc_body(in_ref, pos_ref, out_ref):
    out_ref[...] = in_ref[...] + pos_ref[...][:, None, :]


def _tc_add_head(inp, pos, s_tc, bs):
    S, B, E = inp.shape
    grid = (s_tc // bs,)
    return pl.pallas_call(
        _tc_body,
        grid=grid,
        in_specs=[
            pl.BlockSpec((bs, B, E), lambda i: (i, 0, 0)),
            pl.BlockSpec((bs, E), lambda i: (i, 0)),
        ],
        out_specs=pl.BlockSpec((bs, B, E), lambda i: (i, 0, 0)),
        out_shape=jax.ShapeDtypeStruct((S, B, E), inp.dtype),
    )(inp, pos)


def kernel(input, pos_table):
    S, B, E = input.shape
    pos = pos_table[:S]
    S_SC = 256
    S_TC = S - S_SC
    sc_out = _sc_add_tail(input, pos, S_TC, S_SC)
    tc_out = _tc_add_head(input, pos, S_TC, 496)
    return lax.dynamic_update_slice(tc_out, sc_out, (S_TC, 0, 0))
